# baseline (speedup 1.0000x reference)
import functools

import jax
import jax.numpy as jnp
from jax import lax
from jax.experimental import pallas as pl
from jax.experimental.pallas import tpu as pltpu
from jax.experimental.pallas import tpu_sc as plsc

_FREQ_W = 0.3
_IMP_W = 0.4
_TIME_W = 0.3
_NW = 32
_L = 16


def _mean_body(x_ref, o_ref):
    x = x_ref[0]
    o_ref[0, 0] = jnp.sum(x, axis=-1) * jnp.float32(1.0 / 128.0)


def _row_means(importance):
    b, h = importance.shape
    g = 4
    rows = b // g
    x3 = importance.reshape(g, rows, h)
    out = pl.pallas_call(
        _mean_body,
        grid=(g,),
        in_specs=[pl.BlockSpec((1, rows, h), lambda i: (i, 0, 0))],
        out_specs=pl.BlockSpec((1, 1, rows), lambda i: (i, 0, 0)),
        out_shape=jax.ShapeDtypeStruct((g, 1, rows), jnp.float32),
    )(x3)
    return out.reshape(b)


def _make_scatter_kernel(c, cp, b):
    r = cp // _NW
    groups = b // _L
    last = _NW - 1
    r_last = c - last * r
    mesh = plsc.VectorSubcoreMesh(core_axis_name="c", subcore_axis_name="s")

    @functools.partial(
        pl.kernel,
        mesh=mesh,
        compiler_params=pltpu.CompilerParams(needs_layout_passes=False),
        out_type=[jax.ShapeDtypeStruct((cp,), jnp.float32)] * 3,
        scratch_types=[
            pltpu.VMEM((b,), jnp.int32),
            pltpu.VMEM((b,), jnp.float32),
            pltpu.VMEM((_L,), jnp.float32),
            pltpu.VMEM((r,), jnp.float32),
            pltpu.VMEM((r,), jnp.float32),
            pltpu.VMEM((r,), jnp.float32),
            pltpu.VMEM((r,), jnp.float32),
            pltpu.SemaphoreType.DMA,
            pltpu.SemaphoreType.DMA,
        ],
    )
    def scatter_kernel(idx_hbm, mean_hbm, gt_hbm, t_hbm, f_hbm, i_hbm,
                       t_out, f_out, i_out,
                       idx_v, mean_v, gt_v, t_v, f_v, f_old_v, i_v,
                       in_sem, out_sem):
        wid = lax.axis_index("s") * 2 + lax.axis_index("c")
        lo = wid * r
        off = jnp.minimum(lo, c - r)
        copies = [
            pltpu.async_copy(idx_hbm, idx_v, in_sem),
            pltpu.async_copy(mean_hbm, mean_v, in_sem),
            pltpu.async_copy(gt_hbm, gt_v, in_sem),
            pltpu.async_copy(t_hbm.at[pl.ds(off, r)], t_v, in_sem),
            pltpu.async_copy(f_hbm.at[pl.ds(off, r)], f_v, in_sem),
            pltpu.async_copy(f_hbm.at[pl.ds(off, r)], f_old_v, in_sem),
            pltpu.async_copy(i_hbm.at[pl.ds(off, r)], i_v, in_sem),
        ]
        for cp_ in copies:
            cp_.wait()
        gt16 = gt_v[...]

        def body(g, carry):
            ii = idx_v[pl.ds(g * _L, _L)]
            m = (ii >= lo) & (ii < lo + r)

            @pl.when(jnp.any(m))
            def _():
                locs = jnp.where(m, ii - off, 0)
                plsc.store_scatter(t_v, [locs], gt16, mask=m)
                old = plsc.load_gather(f_old_v, [locs], mask=m)
                plsc.store_scatter(f_v, [locs], old + 1.0, mask=m)
                mu = mean_v[pl.ds(g * _L, _L)]
                plsc.store_scatter(i_v, [locs], mu, mask=m)

            return carry

        lax.fori_loop(0, groups, body, 0, unroll=4)

        @pl.when(wid < last)
        def _():
            o1 = pltpu.async_copy(t_v, t_out.at[pl.ds(lo, r)], out_sem)
            o2 = pltpu.async_copy(f_v, f_out.at[pl.ds(lo, r)], out_sem)
            o3 = pltpu.async_copy(i_v, i_out.at[pl.ds(lo, r)], out_sem)
            o1.wait()
            o2.wait()
            o3.wait()

        @pl.when(wid == last)
        def _():
            ll = last * r
            sk = r - r_last
            o1 = pltpu.async_copy(
                t_v.at[pl.ds(sk, r_last)], t_out.at[pl.ds(ll, r_last)],
                out_sem)
            o2 = pltpu.async_copy(
                f_v.at[pl.ds(sk, r_last)], f_out.at[pl.ds(ll, r_last)],
                out_sem)
            o3 = pltpu.async_copy(
                i_v.at[pl.ds(sk, r_last)], i_out.at[pl.ds(ll, r_last)],
                out_sem)
            o1.wait()
            o2.wait()
            o3.wait()

    return scatter_kernel


def _make_select_body(c, k, rows, mrows):
    def body(denom_ref, t_ref, f_ref, i_ref, p_ref, m_ref):
        t = t_ref[...]
        f = f_ref[...]
        im = i_ref[...]
        row = lax.broadcasted_iota(jnp.int32, (rows, 128), 0)
        col = lax.broadcasted_iota(jnp.int32, (rows, 128), 1)
        flat = row * 128 + col
        valid = flat < c

        zero = jnp.float32(0.0)
        fmax = jnp.max(jnp.where(valid, f, zero))
        imax = jnp.max(jnp.where(valid, im, zero))
        ts = t / denom_ref[0, 0]
        fs = f / (fmax + jnp.float32(1e-8))
        isc = im / (imax + jnp.float32(1e-8))
        p = jnp.float32(_TIME_W) * ts + jnp.float32(_FREQ_W) * fs \
            + jnp.float32(_IMP_W) * isc
        p = jnp.where(valid, p, jnp.float32(jnp.inf))
        p_ref[...] = p
        pbits = lax.bitcast_convert_type(p, jnp.int32)

        def cnt_le(bound):
            return jnp.sum((pbits <= bound).astype(jnp.int32))

        maxfinite = jnp.int32(0x7F7FFFFF)

        def bs_body(_, state):
            lo_, hi_ = state
            mid = (lo_ + hi_) >> 1
            take = cnt_le(mid) >= k
            return jnp.where(take, lo_, mid), jnp.where(take, mid, hi_)

        _, thr = lax.fori_loop(0, 31, bs_body, (jnp.int32(-1), maxfinite))
        c1 = cnt_le(thr - 1)
        need = k - c1
        eq = (pbits == thr) & valid

        def cnt2(bound):
            return jnp.sum((eq & (flat <= bound)).astype(jnp.int32))

        def bs2_body(_, state):
            lo_, hi_ = state
            mid = (lo_ + hi_) >> 1
            take = cnt2(mid) >= need
            return jnp.where(take, lo_, mid), jnp.where(take, mid, hi_)

        _, jthr = lax.fori_loop(0, 18, bs2_body,
                                (jnp.int32(-1), jnp.int32(rows * 128)))
        sel = (pbits < thr) | (eq & (flat <= jthr))
        m_ref[0:rows] = sel.astype(jnp.int32)
        m_ref[rows:mrows] = jnp.zeros((mrows - rows, 128), jnp.int32)

    return body


def kernel(keys, values, indices, importance, access_times, access_frequency,
           importance_scores, global_time):
    cache_len = keys.shape[0]
    c = access_times.shape[0]
    if cache_len <= c:
        return jnp.zeros((cache_len,), dtype=jnp.bool_)
    b = indices.shape[0]
    k = cache_len - c
    del keys, values

    means = importance if importance.ndim == 1 else _row_means(importance)

    cp = ((c + 1023) // 1024) * 1024
    gt_f = jnp.asarray(global_time).astype(jnp.float32)
    gt_vec = jnp.broadcast_to(gt_f, (_L,))
    idx_i32 = indices.astype(jnp.int32)

    scatter = _make_scatter_kernel(c, cp, b)
    t_new, f_new, i_new = scatter(idx_i32, means, gt_vec, access_times,
                                  access_frequency, importance_scores)

    rows = cp // 128
    mrows = cache_len // 128
    denom_t = (jnp.asarray(global_time) + 1).astype(jnp.float32) \
        + jnp.float32(1e-8)
    denom_t = denom_t.reshape(1, 1)
    p_grid, m_grid = pl.pallas_call(
        _make_select_body(c, k, rows, mrows),
        in_specs=[
            pl.BlockSpec(memory_space=pltpu.SMEM),
            pl.BlockSpec((rows, 128), lambda: (0, 0)),
            pl.BlockSpec((rows, 128), lambda: (0, 0)),
            pl.BlockSpec((rows, 128), lambda: (0, 0)),
        ],
        out_specs=[
            pl.BlockSpec((rows, 128), lambda: (0, 0)),
            pl.BlockSpec((mrows, 128), lambda: (0, 0)),
        ],
        out_shape=[
            jax.ShapeDtypeStruct((rows, 128), jnp.float32),
            jax.ShapeDtypeStruct((mrows, 128), jnp.int32),
        ],
    )(denom_t, t_new.reshape(rows, 128), f_new.reshape(rows, 128),
      i_new.reshape(rows, 128))

    priority = p_grid.reshape(cp)[:c]
    evict_mask = m_grid.reshape(cache_len).astype(jnp.bool_)
    return (evict_mask, priority, t_new[:c], f_new[:c], i_new[:c])


# speedup vs baseline: 1.2885x; 1.2885x over previous
import functools

import jax
import jax.numpy as jnp
from jax import lax
from jax.experimental import pallas as pl
from jax.experimental.pallas import tpu as pltpu
from jax.experimental.pallas import tpu_sc as plsc

_FREQ_W = 0.3
_IMP_W = 0.4
_TIME_W = 0.3
_NW = 32
_L = 16


def _mean_body(x_ref, o_ref):
    x = x_ref[0]
    o_ref[0, 0] = jnp.sum(x, axis=-1) * jnp.float32(1.0 / 128.0)


def _row_means(importance):
    b, h = importance.shape
    g = 4
    rows = b // g
    x3 = importance.reshape(g, rows, h)
    out = pl.pallas_call(
        _mean_body,
        grid=(g,),
        in_specs=[pl.BlockSpec((1, rows, h), lambda i: (i, 0, 0))],
        out_specs=pl.BlockSpec((1, 1, rows), lambda i: (i, 0, 0)),
        out_shape=jax.ShapeDtypeStruct((g, 1, rows), jnp.float32),
    )(x3)
    return out.reshape(b)


def _make_scatter_kernel(c, cp, b):
    r = cp // _NW
    groups = b // _L
    last = _NW - 1
    r_last = c - last * r
    mesh = plsc.VectorSubcoreMesh(core_axis_name="c", subcore_axis_name="s")

    @functools.partial(
        pl.kernel,
        mesh=mesh,
        compiler_params=pltpu.CompilerParams(needs_layout_passes=False),
        out_type=[jax.ShapeDtypeStruct((cp,), jnp.float32)] * 3,
        scratch_types=[
            pltpu.VMEM((b,), jnp.int32),
            pltpu.VMEM((b,), jnp.float32),
            pltpu.VMEM((_L,), jnp.float32),
            pltpu.VMEM((r,), jnp.float32),
            pltpu.VMEM((r,), jnp.float32),
            pltpu.VMEM((r,), jnp.float32),
            pltpu.VMEM((r,), jnp.float32),
            pltpu.SemaphoreType.DMA,
            pltpu.SemaphoreType.DMA,
        ],
    )
    def scatter_kernel(idx_hbm, mean_hbm, gt_hbm, t_hbm, f_hbm, i_hbm,
                       t_out, f_out, i_out,
                       idx_v, mean_v, gt_v, t_v, f_v, delta_v, i_v,
                       in_sem, out_sem):
        wid = lax.axis_index("s") * 2 + lax.axis_index("c")
        lo = wid * r
        off = jnp.minimum(lo, c - r)
        copies = [
            pltpu.async_copy(idx_hbm, idx_v, in_sem),
            pltpu.async_copy(mean_hbm, mean_v, in_sem),
            pltpu.async_copy(gt_hbm, gt_v, in_sem),
            pltpu.async_copy(t_hbm.at[pl.ds(off, r)], t_v, in_sem),
            pltpu.async_copy(f_hbm.at[pl.ds(off, r)], f_v, in_sem),
            pltpu.async_copy(i_hbm.at[pl.ds(off, r)], i_v, in_sem),
        ]
        zero16 = jnp.zeros((_L,), jnp.float32)
        one16 = jnp.ones((_L,), jnp.float32)

        def zbody(j, carry):
            delta_v[pl.ds(j * _L, _L)] = zero16
            return carry

        lax.fori_loop(0, r // _L, zbody, 0, unroll=8)
        for cp_ in copies:
            cp_.wait()
        gt16 = gt_v[...]

        def body(g, carry):
            ii = idx_v[pl.ds(g * _L, _L)]
            m = (ii >= lo) & (ii < lo + r)
            locs = jnp.where(m, ii - off, 0)
            plsc.store_scatter(delta_v, [locs], one16, mask=m)
            mu = mean_v[pl.ds(g * _L, _L)]
            plsc.store_scatter(i_v, [locs], mu, mask=m)
            return carry

        lax.fori_loop(0, groups, body, 0, unroll=8)

        def abody(j, carry):
            sl = pl.ds(j * _L, _L)
            d = delta_v[sl]
            hit = d > 0.0
            t_v[sl] = jnp.where(hit, gt16, t_v[sl])
            f_v[sl] = f_v[sl] + d
            return carry

        lax.fori_loop(0, r // _L, abody, 0, unroll=4)

        @pl.when(wid < last)
        def _():
            o1 = pltpu.async_copy(t_v, t_out.at[pl.ds(lo, r)], out_sem)
            o2 = pltpu.async_copy(f_v, f_out.at[pl.ds(lo, r)], out_sem)
            o3 = pltpu.async_copy(i_v, i_out.at[pl.ds(lo, r)], out_sem)
            o1.wait()
            o2.wait()
            o3.wait()

        @pl.when(wid == last)
        def _():
            ll = last * r
            sk = r - r_last
            o1 = pltpu.async_copy(
                t_v.at[pl.ds(sk, r_last)], t_out.at[pl.ds(ll, r_last)],
                out_sem)
            o2 = pltpu.async_copy(
                f_v.at[pl.ds(sk, r_last)], f_out.at[pl.ds(ll, r_last)],
                out_sem)
            o3 = pltpu.async_copy(
                i_v.at[pl.ds(sk, r_last)], i_out.at[pl.ds(ll, r_last)],
                out_sem)
            o1.wait()
            o2.wait()
            o3.wait()

    return scatter_kernel


def _make_select_body(c, k, rows, mrows):
    def body(denom_ref, t_ref, f_ref, i_ref, p_ref, m_ref):
        t = t_ref[...]
        f = f_ref[...]
        im = i_ref[...]
        row = lax.broadcasted_iota(jnp.int32, (rows, 128), 0)
        col = lax.broadcasted_iota(jnp.int32, (rows, 128), 1)
        flat = row * 128 + col
        valid = flat < c

        zero = jnp.float32(0.0)
        fmax = jnp.max(jnp.where(valid, f, zero))
        imax = jnp.max(jnp.where(valid, im, zero))
        ts = t / denom_ref[0, 0]
        fs = f / (fmax + jnp.float32(1e-8))
        isc = im / (imax + jnp.float32(1e-8))
        p = jnp.float32(_TIME_W) * ts + jnp.float32(_FREQ_W) * fs \
            + jnp.float32(_IMP_W) * isc
        p = jnp.where(valid, p, jnp.float32(jnp.inf))
        p_ref[...] = p
        pbits = lax.bitcast_convert_type(p, jnp.int32)

        def cnt_le(bound):
            return jnp.sum((pbits <= bound).astype(jnp.int32))

        maxfinite = jnp.int32(0x7F7FFFFF)

        def bs_body(_, state):
            lo_, hi_ = state
            mid = (lo_ + hi_) >> 1
            take = cnt_le(mid) >= k
            return jnp.where(take, lo_, mid), jnp.where(take, mid, hi_)

        _, thr = lax.fori_loop(0, 31, bs_body, (jnp.int32(-1), maxfinite))
        c1 = cnt_le(thr - 1)
        need = k - c1
        eq = (pbits == thr) & valid

        def cnt2(bound):
            return jnp.sum((eq & (flat <= bound)).astype(jnp.int32))

        def bs2_body(_, state):
            lo_, hi_ = state
            mid = (lo_ + hi_) >> 1
            take = cnt2(mid) >= need
            return jnp.where(take, lo_, mid), jnp.where(take, mid, hi_)

        _, jthr = lax.fori_loop(0, 18, bs2_body,
                                (jnp.int32(-1), jnp.int32(rows * 128)))
        sel = (pbits < thr) | (eq & (flat <= jthr))
        m_ref[0:rows] = sel.astype(jnp.int32)
        m_ref[rows:mrows] = jnp.zeros((mrows - rows, 128), jnp.int32)

    return body


def kernel(keys, values, indices, importance, access_times, access_frequency,
           importance_scores, global_time):
    cache_len = keys.shape[0]
    c = access_times.shape[0]
    if cache_len <= c:
        return jnp.zeros((cache_len,), dtype=jnp.bool_)
    b = indices.shape[0]
    k = cache_len - c
    del keys, values

    means = importance if importance.ndim == 1 else _row_means(importance)

    cp = ((c + 1023) // 1024) * 1024
    gt_f = jnp.asarray(global_time).astype(jnp.float32)
    gt_vec = jnp.broadcast_to(gt_f, (_L,))
    idx_i32 = indices.astype(jnp.int32)

    scatter = _make_scatter_kernel(c, cp, b)
    t_new, f_new, i_new = scatter(idx_i32, means, gt_vec, access_times,
                                  access_frequency, importance_scores)

    rows = cp // 128
    mrows = cache_len // 128
    denom_t = (jnp.asarray(global_time) + 1).astype(jnp.float32) \
        + jnp.float32(1e-8)
    denom_t = denom_t.reshape(1, 1)
    p_grid, m_grid = pl.pallas_call(
        _make_select_body(c, k, rows, mrows),
        in_specs=[
            pl.BlockSpec(memory_space=pltpu.SMEM),
            pl.BlockSpec((rows, 128), lambda: (0, 0)),
            pl.BlockSpec((rows, 128), lambda: (0, 0)),
            pl.BlockSpec((rows, 128), lambda: (0, 0)),
        ],
        out_specs=[
            pl.BlockSpec((rows, 128), lambda: (0, 0)),
            pl.BlockSpec((mrows, 128), lambda: (0, 0)),
        ],
        out_shape=[
            jax.ShapeDtypeStruct((rows, 128), jnp.float32),
            jax.ShapeDtypeStruct((mrows, 128), jnp.int32),
        ],
    )(denom_t, t_new.reshape(rows, 128), f_new.reshape(rows, 128),
      i_new.reshape(rows, 128))

    priority = p_grid.reshape(cp)[:c]
    evict_mask = m_grid.reshape(cache_len).astype(jnp.bool_)
    return (evict_mask, priority, t_new[:c], f_new[:c], i_new[:c])


# speedup vs baseline: 1.3824x; 1.0729x over previous
import functools

import jax
import jax.numpy as jnp
from jax import lax
from jax.experimental import pallas as pl
from jax.experimental.pallas import tpu as pltpu
from jax.experimental.pallas import tpu_sc as plsc

_FREQ_W = 0.3
_IMP_W = 0.4
_TIME_W = 0.3
_NW = 32
_L = 16


def _mean_body(x_ref, o_ref):
    x = x_ref[0]
    o_ref[0, 0] = jnp.sum(x, axis=-1) * jnp.float32(1.0 / 128.0)


def _row_means(importance):
    b, h = importance.shape
    g = 4
    rows = b // g
    x3 = importance.reshape(g, rows, h)
    out = pl.pallas_call(
        _mean_body,
        grid=(g,),
        in_specs=[pl.BlockSpec((1, rows, h), lambda i: (i, 0, 0))],
        out_specs=pl.BlockSpec((1, 1, rows), lambda i: (i, 0, 0)),
        out_shape=jax.ShapeDtypeStruct((g, 1, rows), jnp.float32),
    )(x3)
    return out.reshape(b)


def _make_scatter_kernel(c, cp, b):
    r = cp // _NW
    groups = b // _L
    last = _NW - 1
    r_last = c - last * r
    mesh = plsc.VectorSubcoreMesh(core_axis_name="c", subcore_axis_name="s")

    @functools.partial(
        pl.kernel,
        mesh=mesh,
        compiler_params=pltpu.CompilerParams(needs_layout_passes=False),
        out_type=[jax.ShapeDtypeStruct((cp,), jnp.float32)] * 3,
        scratch_types=[
            pltpu.VMEM((b,), jnp.int32),
            pltpu.VMEM((b,), jnp.float32),
            pltpu.VMEM((_L,), jnp.float32),
            pltpu.VMEM((r,), jnp.float32),
            pltpu.VMEM((r,), jnp.float32),
            pltpu.VMEM((r,), jnp.float32),
            pltpu.VMEM((r,), jnp.float32),
            pltpu.SemaphoreType.DMA,
            pltpu.SemaphoreType.DMA,
        ],
    )
    def scatter_kernel(idx_hbm, mean_hbm, gt_hbm, t_hbm, f_hbm, i_hbm,
                       t_out, f_out, i_out,
                       idx_v, mean_v, gt_v, t_v, f_v, delta_v, i_v,
                       in_sem, out_sem):
        wid = lax.axis_index("s") * 2 + lax.axis_index("c")
        lo = wid * r
        off = jnp.minimum(lo, c - r)
        copies = [
            pltpu.async_copy(idx_hbm, idx_v, in_sem),
            pltpu.async_copy(mean_hbm, mean_v, in_sem),
            pltpu.async_copy(gt_hbm, gt_v, in_sem),
            pltpu.async_copy(t_hbm.at[pl.ds(off, r)], t_v, in_sem),
            pltpu.async_copy(f_hbm.at[pl.ds(off, r)], f_v, in_sem),
            pltpu.async_copy(i_hbm.at[pl.ds(off, r)], i_v, in_sem),
        ]
        zero16 = jnp.zeros((_L,), jnp.float32)
        one16 = jnp.ones((_L,), jnp.float32)

        def zbody(j, carry):
            delta_v[pl.ds(j * _L, _L)] = zero16
            return carry

        lax.fori_loop(0, r // _L, zbody, 0, unroll=8)
        for cp_ in copies:
            cp_.wait()
        gt16 = gt_v[...]

        def body(g, carry):
            ii = idx_v[pl.ds(g * _L, _L)]
            m = (ii >= lo) & (ii < lo + r)
            locs = jnp.where(m, ii - off, 0)
            plsc.store_scatter(delta_v, [locs], one16, mask=m)
            mu = mean_v[pl.ds(g * _L, _L)]
            plsc.store_scatter(i_v, [locs], mu, mask=m)
            return carry

        lax.fori_loop(0, groups, body, 0, unroll=8)

        def abody(j, carry):
            sl = pl.ds(j * _L, _L)
            d = delta_v[sl]
            hit = d > 0.0
            t_v[sl] = jnp.where(hit, gt16, t_v[sl])
            f_v[sl] = f_v[sl] + d
            return carry

        lax.fori_loop(0, r // _L, abody, 0, unroll=4)

        @pl.when(wid < last)
        def _():
            o1 = pltpu.async_copy(t_v, t_out.at[pl.ds(lo, r)], out_sem)
            o2 = pltpu.async_copy(f_v, f_out.at[pl.ds(lo, r)], out_sem)
            o3 = pltpu.async_copy(i_v, i_out.at[pl.ds(lo, r)], out_sem)
            o1.wait()
            o2.wait()
            o3.wait()

        @pl.when(wid == last)
        def _():
            ll = last * r
            sk = r - r_last
            o1 = pltpu.async_copy(
                t_v.at[pl.ds(sk, r_last)], t_out.at[pl.ds(ll, r_last)],
                out_sem)
            o2 = pltpu.async_copy(
                f_v.at[pl.ds(sk, r_last)], f_out.at[pl.ds(ll, r_last)],
                out_sem)
            o3 = pltpu.async_copy(
                i_v.at[pl.ds(sk, r_last)], i_out.at[pl.ds(ll, r_last)],
                out_sem)
            o1.wait()
            o2.wait()
            o3.wait()

    return scatter_kernel


def _make_select_body(c, k, rows, mrows):
    def body(denom_ref, t_ref, f_ref, i_ref, p_ref, m_ref):
        t = t_ref[...]
        f = f_ref[...]
        im = i_ref[...]
        row = lax.broadcasted_iota(jnp.int32, (rows, 128), 0)
        col = lax.broadcasted_iota(jnp.int32, (rows, 128), 1)
        flat = row * 128 + col
        valid = flat < c

        zero = jnp.float32(0.0)
        fmax = jnp.max(jnp.where(valid, f, zero))
        imax = jnp.max(jnp.where(valid, im, zero))
        ts = t / denom_ref[0, 0]
        fs = f / (fmax + jnp.float32(1e-8))
        isc = im / (imax + jnp.float32(1e-8))
        p = jnp.float32(_TIME_W) * ts + jnp.float32(_FREQ_W) * fs \
            + jnp.float32(_IMP_W) * isc
        p = jnp.where(valid, p, jnp.float32(jnp.inf))
        p_ref[...] = p
        pbits = lax.bitcast_convert_type(p, jnp.int32)

        def cnt_le(bound):
            return jnp.sum((pbits <= bound).astype(jnp.int32))

        maxfinite = jnp.int32(0x7F7FFFFF)

        def bs_body(_, state):
            lo_, hi_ = state
            mid = (lo_ + hi_) >> 1
            take = cnt_le(mid) >= k
            return jnp.where(take, lo_, mid), jnp.where(take, mid, hi_)

        top = jnp.int32(0x3F800000)
        _, thr = lax.fori_loop(0, 30, bs_body, (jnp.int32(-1), top))
        c1 = cnt_le(thr - 1)
        ctot = cnt_le(thr)
        need = k - c1
        eq = (pbits == thr) & valid

        def cnt2(bound):
            return jnp.sum((eq & (flat <= bound)).astype(jnp.int32))

        def bs2_body(_, state):
            lo_, hi_ = state
            mid = (lo_ + hi_) >> 1
            take = cnt2(mid) >= need
            return jnp.where(take, lo_, mid), jnp.where(take, mid, hi_)

        def tie_break():
            _, j = lax.fori_loop(0, 18, bs2_body,
                                 (jnp.int32(-1), jnp.int32(rows * 128)))
            return j

        jthr = lax.cond(ctot - c1 == need,
                        lambda: jnp.int32(rows * 128), tie_break)
        sel = (pbits < thr) | (eq & (flat <= jthr))
        m_ref[0:rows] = sel.astype(jnp.int32)
        m_ref[rows:mrows] = jnp.zeros((mrows - rows, 128), jnp.int32)

    return body


def kernel(keys, values, indices, importance, access_times, access_frequency,
           importance_scores, global_time):
    cache_len = keys.shape[0]
    c = access_times.shape[0]
    if cache_len <= c:
        return jnp.zeros((cache_len,), dtype=jnp.bool_)
    b = indices.shape[0]
    k = cache_len - c
    del keys, values

    means = importance if importance.ndim == 1 else _row_means(importance)

    cp = ((c + 1023) // 1024) * 1024
    gt_f = jnp.asarray(global_time).astype(jnp.float32)
    gt_vec = jnp.broadcast_to(gt_f, (_L,))
    idx_i32 = indices.astype(jnp.int32)

    scatter = _make_scatter_kernel(c, cp, b)
    t_new, f_new, i_new = scatter(idx_i32, means, gt_vec, access_times,
                                  access_frequency, importance_scores)

    rows = cp // 128
    mrows = cache_len // 128
    denom_t = (jnp.asarray(global_time) + 1).astype(jnp.float32) \
        + jnp.float32(1e-8)
    denom_t = denom_t.reshape(1, 1)
    p_grid, m_grid = pl.pallas_call(
        _make_select_body(c, k, rows, mrows),
        in_specs=[
            pl.BlockSpec(memory_space=pltpu.SMEM),
            pl.BlockSpec((rows, 128), lambda: (0, 0)),
            pl.BlockSpec((rows, 128), lambda: (0, 0)),
            pl.BlockSpec((rows, 128), lambda: (0, 0)),
        ],
        out_specs=[
            pl.BlockSpec((rows, 128), lambda: (0, 0)),
            pl.BlockSpec((mrows, 128), lambda: (0, 0)),
        ],
        out_shape=[
            jax.ShapeDtypeStruct((rows, 128), jnp.float32),
            jax.ShapeDtypeStruct((mrows, 128), jnp.int32),
        ],
    )(denom_t, t_new.reshape(rows, 128), f_new.reshape(rows, 128),
      i_new.reshape(rows, 128))

    priority = p_grid.reshape(cp)[:c]
    evict_mask = m_grid.reshape(cache_len).astype(jnp.bool_)
    return (evict_mask, priority, t_new[:c], f_new[:c], i_new[:c])
